# Initial kernel scaffold; baseline (speedup 1.0000x reference)
import jax
import jax.numpy as jnp
from jax.experimental import pallas as pl


def kernel(feats, segment_ids, weight, bias, mean_scale):
    raise NotImplementedError("write your pallas kernel here")


# speedup vs baseline: 2.6122x; 2.6122x over previous
import functools

import jax
import jax.numpy as jnp
from jax import lax
from jax.experimental import pallas as pl
from jax.experimental.pallas import tpu as pltpu
from jax.experimental.pallas import tpu_sc as plsc

N = 100000
D = 128
NSEG = 512
EPS = 1e-05

NC = 2
NS = 16
NW = NC * NS
BLK = 200
SB = N // BLK
MAXB = 16
IDS_LEN = 3232
ACC_ROWS = 640

_mesh = plsc.VectorSubcoreMesh(core_axis_name="c", subcore_axis_name="s")


def _iota16():
    return lax.iota(jnp.int32, 16)


def _id_at(ids_v, row):
    return ids_v[pl.ds(row, 16)][0]


def _worker_range(gid):
    sb0 = (SB * gid) // NW
    sb1 = (SB * (gid + 1)) // NW
    start = sb0 * BLK
    delta = start % 16
    base_al = pl.multiple_of(start - delta, 8)
    return sb0, sb1, base_al, delta


@functools.partial(
    pl.kernel,
    out_type=(
        jax.ShapeDtypeStruct((NC * NSEG * D,), jnp.float32),
        jax.ShapeDtypeStruct((NC * NSEG * D,), jnp.float32),
        jax.ShapeDtypeStruct((NC * NSEG * 16,), jnp.float32),
    ),
    mesh=_mesh,
    scratch_types=[
        pltpu.VMEM((IDS_LEN,), jnp.int32),
        pltpu.VMEM((BLK * D,), jnp.float32),
        pltpu.VMEM((D,), jnp.float32),
        pltpu.VMEM((D,), jnp.float32),
        pltpu.VMEM((16,), jnp.float32),
        pltpu.VMEM((D,), jnp.int32),
        pltpu.VMEM((16,), jnp.int32),
        pltpu.VMEM((40 * D,), jnp.float32),
        pltpu.VMEM_SHARED((ACC_ROWS * D,), jnp.float32),
        pltpu.VMEM_SHARED((ACC_ROWS * D,), jnp.float32),
        pltpu.VMEM_SHARED((ACC_ROWS * 16,), jnp.float32),
    ],
)
def _sc_accumulate(feats_hbm, ids_hbm, psum_hbm, psq_hbm, pcnt_hbm,
                   ids_v, buf, st_sum, st_sq, st_cnt, st_idx, st_idx16,
                   zb, sh_sum, sh_sq, sh_cnt):
    c = lax.axis_index("c")
    s = lax.axis_index("s")
    gid = c * NS + s
    zvec = jnp.zeros((16,), jnp.float32)

    def _zchunk(i, _):
        zb[pl.ds(i * 16, 16)] = zvec
        return 0
    lax.fori_loop(0, 40 * D // 16, _zchunk, 0)
    z0 = pl.multiple_of((40 * D) * s, 8)
    pltpu.sync_copy(zb, sh_sum.at[pl.ds(z0, 40 * D)])
    pltpu.sync_copy(zb, sh_sq.at[pl.ds(z0, 40 * D)])
    z16 = pl.multiple_of((40 * 16) * s, 8)
    pltpu.sync_copy(zb.at[pl.ds(0, 40 * 16)], sh_cnt.at[pl.ds(z16, 40 * 16)])
    plsc.subcore_barrier()

    sb0, sb1, base_al, delta = _worker_range(gid)
    pltpu.sync_copy(ids_hbm.at[pl.ds(base_al, IDS_LEN)], ids_v)

    def _flush(cur, cnt, accs, sqs):
        iota = _iota16()
        for j in range(D // 16):
            st_sum[pl.ds(16 * j, 16)] = accs[j]
            st_sq[pl.ds(16 * j, 16)] = sqs[j]
            st_idx[pl.ds(16 * j, 16)] = iota + (cur * D + 16 * j)
        st_cnt[pl.ds(0, 16)] = jnp.full((16,), cnt, jnp.float32)
        st_idx16[pl.ds(0, 16)] = iota + cur * 16
        pltpu.sync_copy(st_sum, sh_sum.at[st_idx], add=True)
        pltpu.sync_copy(st_sq, sh_sq.at[st_idx], add=True)
        pltpu.sync_copy(st_cnt, sh_cnt.at[st_idx16], add=True)

    def _block(b, carry):
        boff = pl.multiple_of((sb0 + b) * (BLK * D), 8)
        pltpu.sync_copy(feats_hbm.at[pl.ds(boff, BLK * D)], buf)

        def _row(r, carry):
            cur, cnt, *vs = carry
            accs, sqs = list(vs[:8]), list(vs[8:])
            sid = _id_at(ids_v, b * BLK + r + delta)
            chg = sid != cur

            @pl.when(chg)
            def _():
                _flush(cur, cnt, accs, sqs)

            keep = jnp.where(chg, 0.0, 1.0)
            cur = jnp.where(chg, sid, cur)
            cnt = cnt * keep + 1.0
            for j in range(D // 16):
                x = buf[pl.ds(r * D + 16 * j, 16)]
                accs[j] = accs[j] * keep + x
                sqs[j] = sqs[j] * keep + x * x
            return (cur, cnt, *accs, *sqs)

        return lax.fori_loop(0, BLK, _row, carry)

    cur0 = _id_at(ids_v, delta)
    init = (cur0, jnp.float32(0), *([zvec] * 8), *([zvec] * 8))
    cur, cnt, *vs = lax.fori_loop(0, sb1 - sb0, _block, init)
    _flush(cur, cnt, vs[:8], vs[8:])

    plsc.subcore_barrier()
    p0 = pl.multiple_of((32 * D) * s, 8)
    o0 = pl.multiple_of(c * (NSEG * D) + (32 * D) * s, 8)
    pltpu.sync_copy(sh_sum.at[pl.ds(p0, 32 * D)],
                    psum_hbm.at[pl.ds(o0, 32 * D)])
    pltpu.sync_copy(sh_sq.at[pl.ds(p0, 32 * D)],
                    psq_hbm.at[pl.ds(o0, 32 * D)])
    p16 = pl.multiple_of((32 * 16) * s, 8)
    o16 = pl.multiple_of(c * (NSEG * 16) + (32 * 16) * s, 8)
    pltpu.sync_copy(sh_cnt.at[pl.ds(p16, 32 * 16)],
                    pcnt_hbm.at[pl.ds(o16, 32 * 16)])


def _tc_stats(psum, psq, pcnt, weight, bias, mean_scale):
    def body(psum_ref, psq_ref, pcnt_ref, w_ref, b_ref, ms_ref,
             a_ref, c_ref):
        sum_ = psum_ref[0] + psum_ref[1]
        sq = psq_ref[0] + psq_ref[1]
        cnt = pcnt_ref[0, :, 0:1] + pcnt_ref[1, :, 0:1]
        n = jnp.maximum(cnt, 1.0)
        m = sum_ / n
        msc = m * ms_ref[...]
        varsum = jnp.maximum(sq - msc * (2.0 * sum_ - n * msc), 0.0)
        std = jnp.sqrt(varsum / n + EPS)
        a = w_ref[...] / std
        a_ref[...] = a
        c_ref[...] = b_ref[...] - a * msc

    return pl.pallas_call(
        body,
        out_shape=(
            jax.ShapeDtypeStruct((NSEG, D), jnp.float32),
            jax.ShapeDtypeStruct((NSEG, D), jnp.float32),
        ),
    )(psum, psq, pcnt, weight, bias, mean_scale)


@functools.partial(
    pl.kernel,
    out_type=jax.ShapeDtypeStruct((N * D,), jnp.float32),
    mesh=_mesh,
    scratch_types=[
        pltpu.VMEM((IDS_LEN,), jnp.int32),
        pltpu.VMEM((BLK * D,), jnp.float32),
        pltpu.VMEM((BLK * D,), jnp.float32),
        pltpu.VMEM((D,), jnp.float32),
        pltpu.VMEM((D,), jnp.float32),
    ],
)
def _sc_normalize(feats_hbm, ids_hbm, a_hbm, c_hbm, out_hbm,
                  ids_v, buf, obuf, arow, crow):
    c = lax.axis_index("c")
    s = lax.axis_index("s")
    gid = c * NS + s
    sb0, sb1, base_al, delta = _worker_range(gid)
    pltpu.sync_copy(ids_hbm.at[pl.ds(base_al, IDS_LEN)], ids_v)
    zvec = jnp.zeros((16,), jnp.float32)

    def _block(b, carry):
        boff = pl.multiple_of((sb0 + b) * (BLK * D), 8)
        pltpu.sync_copy(feats_hbm.at[pl.ds(boff, BLK * D)], buf)

        def _row(r, cur):
            sid = _id_at(ids_v, b * BLK + r + delta)
            chg = sid != cur

            @pl.when(chg)
            def _():
                soff = pl.multiple_of(sid * D, 8)
                pltpu.sync_copy(a_hbm.at[pl.ds(soff, D)], arow)
                pltpu.sync_copy(c_hbm.at[pl.ds(soff, D)], crow)

            for j in range(D // 16):
                x = buf[pl.ds(r * D + 16 * j, 16)]
                obuf[pl.ds(r * D + 16 * j, 16)] = (
                    arow[pl.ds(16 * j, 16)] * x + crow[pl.ds(16 * j, 16)])
            return jnp.where(chg, sid, cur)

        carry = lax.fori_loop(0, BLK, _row, carry)
        pltpu.sync_copy(obuf, out_hbm.at[pl.ds(boff, BLK * D)])
        return carry

    lax.fori_loop(0, sb1 - sb0, _block, jnp.int32(-1))


def kernel(feats, segment_ids, weight, bias, mean_scale):
    ids = segment_ids.astype(jnp.int32)
    ids_pad = jnp.pad(ids, (0, 128))
    feats_flat = feats.reshape(N * D)
    psum, psq, pcnt = _sc_accumulate(feats_flat, ids_pad)
    psum = psum.reshape(NC, NSEG, D)
    psq = psq.reshape(NC, NSEG, D)
    pcnt = pcnt.reshape(NC, NSEG, 16)
    a, c = _tc_stats(psum, psq, pcnt, weight.reshape(1, D),
                     bias.reshape(1, D), mean_scale.reshape(1, D))
    out = _sc_normalize(feats_flat, ids_pad, a.reshape(NSEG * D),
                        c.reshape(NSEG * D))
    return out.reshape(N, D)


# speedup vs baseline: 3.5049x; 1.3417x over previous
import functools

import jax
import jax.numpy as jnp
from jax import lax
from jax.experimental import pallas as pl
from jax.experimental.pallas import tpu as pltpu
from jax.experimental.pallas import tpu_sc as plsc

N = 100000
D = 128
NSEG = 512
EPS = 1e-05

NC = 2
NS = 16
NW = NC * NS
BLK = 160
NG = BLK // 16
SB = N // BLK
BLKW = BLK * D
IDS_LEN = 3216
ACC_ROWS = 640

_mesh = plsc.VectorSubcoreMesh(core_axis_name="c", subcore_axis_name="s")


def _iota16():
    return lax.iota(jnp.int32, 16)


def _worker_range(gid):
    sb0 = (SB * gid) // NW
    sb1 = (SB * (gid + 1)) // NW
    base_al = pl.multiple_of(sb0 * BLK, 8)
    return sb0, sb1, base_al


@functools.partial(
    pl.kernel,
    out_type=(
        jax.ShapeDtypeStruct((NC * NSEG * D,), jnp.float32),
        jax.ShapeDtypeStruct((NC * NSEG * D,), jnp.float32),
        jax.ShapeDtypeStruct((NC * NSEG * 16,), jnp.float32),
    ),
    mesh=_mesh,
    scratch_types=[
        pltpu.VMEM((IDS_LEN,), jnp.int32),
        pltpu.VMEM((2 * BLKW,), jnp.float32),
        pltpu.VMEM((D,), jnp.float32),
        pltpu.VMEM((D,), jnp.float32),
        pltpu.VMEM((D,), jnp.float32),
        pltpu.VMEM((D,), jnp.float32),
        pltpu.VMEM((16,), jnp.float32),
        pltpu.VMEM((D,), jnp.int32),
        pltpu.VMEM((16,), jnp.int32),
        pltpu.VMEM((40 * D,), jnp.float32),
        pltpu.SemaphoreType.DMA,
        pltpu.VMEM_SHARED((ACC_ROWS * D,), jnp.float32),
        pltpu.VMEM_SHARED((ACC_ROWS * D,), jnp.float32),
        pltpu.VMEM_SHARED((ACC_ROWS * 16,), jnp.float32),
    ],
)
def _sc_accumulate(feats_hbm, ids_hbm, psum_hbm, psq_hbm, pcnt_hbm,
                   ids_v, buf, snap_sum, snap_sq, st_sum, st_sq, st_cnt,
                   st_idx, st_idx16, zb, sem, sh_sum, sh_sq, sh_cnt):
    c = lax.axis_index("c")
    s = lax.axis_index("s")
    gid = c * NS + s
    zvec = jnp.zeros((16,), jnp.float32)

    def _zchunk(i, _):
        zb[pl.ds(i * 16, 16)] = zvec
        return 0
    lax.fori_loop(0, 40 * D // 16, _zchunk, 0)
    z0 = pl.multiple_of((40 * D) * s, 8)
    pltpu.sync_copy(zb, sh_sum.at[pl.ds(z0, 40 * D)])
    pltpu.sync_copy(zb, sh_sq.at[pl.ds(z0, 40 * D)])
    z16 = pl.multiple_of((40 * 16) * s, 8)
    pltpu.sync_copy(zb.at[pl.ds(0, 40 * 16)], sh_cnt.at[pl.ds(z16, 40 * 16)])
    for j in range(D // 16):
        snap_sum[pl.ds(16 * j, 16)] = zvec
        snap_sq[pl.ds(16 * j, 16)] = zvec
    plsc.subcore_barrier()

    sb0, sb1, base_al = _worker_range(gid)
    nblk = sb1 - sb0
    pltpu.sync_copy(ids_hbm.at[pl.ds(base_al, IDS_LEN)], ids_v)

    def _feats_copy(b, slot):
        boff = pl.multiple_of((sb0 + b) * BLKW, 8)
        doff = pl.multiple_of(slot * BLKW, 8)
        return pltpu.make_async_copy(feats_hbm.at[pl.ds(boff, BLKW)],
                                     buf.at[pl.ds(doff, BLKW)], sem)

    _feats_copy(0, 0).start()

    def _flush(cur, cnt, totals, totsq):
        iota = _iota16()
        for j in range(D // 16):
            sl = pl.ds(16 * j, 16)
            st_sum[sl] = totals[j] - snap_sum[sl]
            st_sq[sl] = totsq[j] - snap_sq[sl]
            snap_sum[sl] = totals[j]
            snap_sq[sl] = totsq[j]
            st_idx[sl] = iota + (cur * D + 16 * j)
        st_cnt[pl.ds(0, 16)] = jnp.full((16,), cnt, jnp.float32)
        st_idx16[pl.ds(0, 16)] = iota + cur * 16
        pltpu.sync_copy(st_sum, sh_sum.at[st_idx], add=True)
        pltpu.sync_copy(st_sq, sh_sq.at[st_idx], add=True)
        pltpu.sync_copy(st_cnt, sh_cnt.at[st_idx16], add=True)

    def _block(b, carry):
        _feats_copy(b, b % 2).wait()

        @pl.when(b + 1 < nblk)
        def _():
            _feats_copy(b + 1, (b + 1) % 2).start()

        bb = pl.multiple_of((b % 2) * BLKW, 8)

        def _row(r, carry):
            cur, srows, *vs = carry
            totals, totsq = list(vs[:8]), list(vs[8:])
            row = b * BLK + r
            sid = ids_v[pl.ds(row, 16)][0]
            chg = sid != cur

            @pl.when(chg)
            def _():
                _flush(cur, (row - srows).astype(jnp.float32),
                       totals, totsq)

            cur = jnp.where(chg, sid, cur)
            srows = jnp.where(chg, row, srows)
            for j in range(D // 16):
                x = buf[pl.ds(bb + r * D + 16 * j, 16)]
                totals[j] = totals[j] + x
                totsq[j] = totsq[j] + x * x
            return (cur, srows, *totals, *totsq)

        return lax.fori_loop(0, BLK, _row, carry)

    cur0 = ids_v[pl.ds(0, 16)][0]
    init = (cur0, jnp.int32(0), *([zvec] * 16))
    cur, srows, *vs = lax.fori_loop(0, nblk, _block, init)
    _flush(cur, (nblk * BLK - srows).astype(jnp.float32), vs[:8], vs[8:])

    plsc.subcore_barrier()
    p0 = pl.multiple_of((32 * D) * s, 8)
    o0 = pl.multiple_of(c * (NSEG * D) + (32 * D) * s, 8)
    pltpu.sync_copy(sh_sum.at[pl.ds(p0, 32 * D)],
                    psum_hbm.at[pl.ds(o0, 32 * D)])
    pltpu.sync_copy(sh_sq.at[pl.ds(p0, 32 * D)],
                    psq_hbm.at[pl.ds(o0, 32 * D)])
    p16 = pl.multiple_of((32 * 16) * s, 8)
    o16 = pl.multiple_of(c * (NSEG * 16) + (32 * 16) * s, 8)
    pltpu.sync_copy(sh_cnt.at[pl.ds(p16, 32 * 16)],
                    pcnt_hbm.at[pl.ds(o16, 32 * 16)])


def _tc_stats(psum, psq, pcnt, weight, bias, mean_scale):
    def body(psum_ref, psq_ref, pcnt_ref, w_ref, b_ref, ms_ref,
             a_ref, c_ref):
        sum_ = psum_ref[0] + psum_ref[1]
        sq = psq_ref[0] + psq_ref[1]
        cnt = pcnt_ref[0, :, 0:1] + pcnt_ref[1, :, 0:1]
        n = jnp.maximum(cnt, 1.0)
        m = sum_ / n
        msc = m * ms_ref[...]
        varsum = jnp.maximum(sq - msc * (2.0 * sum_ - n * msc), 0.0)
        std = jnp.sqrt(varsum / n + EPS)
        a = w_ref[...] / std
        a_ref[...] = a
        c_ref[...] = b_ref[...] - a * msc

    return pl.pallas_call(
        body,
        out_shape=(
            jax.ShapeDtypeStruct((NSEG, D), jnp.float32),
            jax.ShapeDtypeStruct((NSEG, D), jnp.float32),
        ),
    )(psum, psq, pcnt, weight, bias, mean_scale)


@functools.partial(
    pl.kernel,
    out_type=jax.ShapeDtypeStruct((N * D,), jnp.float32),
    mesh=_mesh,
    scratch_types=[
        pltpu.VMEM((IDS_LEN,), jnp.int32),
        pltpu.VMEM((2 * BLKW,), jnp.float32),
        pltpu.VMEM((2 * BLKW,), jnp.float32),
        pltpu.VMEM((D,), jnp.float32),
        pltpu.VMEM((D,), jnp.float32),
        pltpu.SemaphoreType.DMA,
        pltpu.SemaphoreType.DMA,
    ],
)
def _sc_normalize(feats_hbm, ids_hbm, a_hbm, c_hbm, out_hbm,
                  ids_v, buf, obuf, arow, crow, sem_in, sem_out):
    c = lax.axis_index("c")
    s = lax.axis_index("s")
    gid = c * NS + s
    sb0, sb1, base_al = _worker_range(gid)
    nblk = sb1 - sb0
    pltpu.sync_copy(ids_hbm.at[pl.ds(base_al, IDS_LEN)], ids_v)

    def _in_copy(b, slot):
        boff = pl.multiple_of((sb0 + b) * BLKW, 8)
        doff = pl.multiple_of(slot * BLKW, 8)
        return pltpu.make_async_copy(feats_hbm.at[pl.ds(boff, BLKW)],
                                     buf.at[pl.ds(doff, BLKW)], sem_in)

    def _out_copy(b, slot):
        boff = pl.multiple_of((sb0 + b) * BLKW, 8)
        soff = pl.multiple_of(slot * BLKW, 8)
        return pltpu.make_async_copy(obuf.at[pl.ds(soff, BLKW)],
                                     out_hbm.at[pl.ds(boff, BLKW)], sem_out)

    _in_copy(0, 0).start()

    def _pull(sid):
        soff = pl.multiple_of(sid * D, 8)
        pltpu.sync_copy(a_hbm.at[pl.ds(soff, D)], arow)
        pltpu.sync_copy(c_hbm.at[pl.ds(soff, D)], crow)

    def _block(b, cur):
        _in_copy(b, b % 2).wait()

        @pl.when(b + 1 < nblk)
        def _():
            _in_copy(b + 1, (b + 1) % 2).start()

        @pl.when(b >= 2)
        def _():
            _out_copy(b - 2, b % 2).wait()

        bb = pl.multiple_of((b % 2) * BLKW, 8)

        def _group(g, cur):
            row0 = b * BLK + g * 16
            idv = ids_v[pl.ds(row0, 16)]
            first = idv[0]
            last = idv[15]
            uniform = first == last

            @pl.when(uniform & (first != cur))
            def _():
                _pull(first)

            def _fast(cur):
                avs = [arow[pl.ds(16 * j, 16)] for j in range(D // 16)]
                cvs = [crow[pl.ds(16 * j, 16)] for j in range(D // 16)]

                def _frow(r, _):
                    o = bb + (g * 16 + r) * D
                    for j in range(D // 16):
                        x = buf[pl.ds(o + 16 * j, 16)]
                        obuf[pl.ds(o + 16 * j, 16)] = avs[j] * x + cvs[j]
                    return 0
                lax.fori_loop(0, 16, _frow, 0)
                return last

            def _slow(cur):
                def _srow(r, cur):
                    sid = ids_v[pl.ds(row0 + r, 16)][0]

                    @pl.when(sid != cur)
                    def _():
                        _pull(sid)

                    o = bb + (g * 16 + r) * D
                    for j in range(D // 16):
                        x = buf[pl.ds(o + 16 * j, 16)]
                        obuf[pl.ds(o + 16 * j, 16)] = (
                            arow[pl.ds(16 * j, 16)] * x
                            + crow[pl.ds(16 * j, 16)])
                    return jnp.where(sid != cur, sid, cur)
                return lax.fori_loop(0, 16, _srow, cur)

            return lax.cond(uniform, _fast, _slow, cur)

        cur = lax.fori_loop(0, NG, _group, cur)
        _out_copy(b, b % 2).start()
        return cur

    lax.fori_loop(0, nblk, _block, jnp.int32(-1))

    @pl.when(nblk >= 2)
    def _():
        _out_copy(nblk - 2, nblk % 2).wait()
    _out_copy(nblk - 1, (nblk - 1) % 2).wait()


def kernel(feats, segment_ids, weight, bias, mean_scale):
    ids = segment_ids.astype(jnp.int32)
    ids_pad = jnp.pad(ids, (0, 128))
    feats_flat = feats.reshape(N * D)
    psum, psq, pcnt = _sc_accumulate(feats_flat, ids_pad)
    psum = psum.reshape(NC, NSEG, D)
    psq = psq.reshape(NC, NSEG, D)
    pcnt = pcnt.reshape(NC, NSEG, 16)
    a, c = _tc_stats(psum, psq, pcnt, weight.reshape(1, D),
                     bias.reshape(1, D), mean_scale.reshape(1, D))
    out = _sc_normalize(feats_flat, ids_pad, a.reshape(NSEG * D),
                        c.reshape(NSEG * D))
    return out.reshape(N, D)


# speedup vs baseline: 4.4631x; 1.2734x over previous
import functools

import jax
import jax.numpy as jnp
from jax import lax
from jax.experimental import pallas as pl
from jax.experimental.pallas import tpu as pltpu
from jax.experimental.pallas import tpu_sc as plsc

N = 100000
D = 128
NSEG = 512
EPS = 1e-05

NC = 2
NS = 16
NW = NC * NS
BLK = 160
NG = BLK // 16
SB = N // BLK
BLKW = BLK * D
IDS_LEN = 3216
ACC_ROWS = 640

_mesh = plsc.VectorSubcoreMesh(core_axis_name="c", subcore_axis_name="s")
_NJ = D // 16


def _iota16():
    return lax.iota(jnp.int32, 16)


def _worker_range(gid):
    sb0 = (SB * gid) // NW
    sb1 = (SB * (gid + 1)) // NW
    base_al = pl.multiple_of(sb0 * BLK, 8)
    return sb0, sb1, base_al


@functools.partial(
    pl.kernel,
    out_type=(
        jax.ShapeDtypeStruct((NC * NSEG * D,), jnp.float32),
        jax.ShapeDtypeStruct((NC * NSEG * D,), jnp.float32),
        jax.ShapeDtypeStruct((NC * NSEG * 16,), jnp.float32),
    ),
    mesh=_mesh,
    scratch_types=[
        pltpu.VMEM((IDS_LEN,), jnp.int32),
        pltpu.VMEM((2 * BLKW,), jnp.float32),
        pltpu.VMEM((D,), jnp.float32),
        pltpu.VMEM((D,), jnp.float32),
        pltpu.VMEM((D,), jnp.float32),
        pltpu.VMEM((D,), jnp.float32),
        pltpu.VMEM((D,), jnp.float32),
        pltpu.VMEM((D,), jnp.float32),
        pltpu.VMEM((16,), jnp.float32),
        pltpu.VMEM((D,), jnp.int32),
        pltpu.VMEM((16,), jnp.int32),
        pltpu.VMEM((40 * D,), jnp.float32),
        pltpu.SemaphoreType.DMA,
        pltpu.VMEM_SHARED((ACC_ROWS * D,), jnp.float32),
        pltpu.VMEM_SHARED((ACC_ROWS * D,), jnp.float32),
        pltpu.VMEM_SHARED((ACC_ROWS * 16,), jnp.float32),
    ],
)
def _sc_accumulate(feats_hbm, ids_hbm, psum_hbm, psq_hbm, pcnt_hbm,
                   ids_v, buf, tot_sum, tot_sq, snap_sum, snap_sq,
                   st_sum, st_sq, st_cnt, st_idx, st_idx16, zb, sem,
                   sh_sum, sh_sq, sh_cnt):
    c = lax.axis_index("c")
    s = lax.axis_index("s")
    gid = c * NS + s
    zvec = jnp.zeros((16,), jnp.float32)

    def _zchunk(i, _):
        zb[pl.ds(i * 16, 16)] = zvec
        return 0
    lax.fori_loop(0, 40 * D // 16, _zchunk, 0)
    z0 = pl.multiple_of((40 * D) * s, 8)
    pltpu.sync_copy(zb, sh_sum.at[pl.ds(z0, 40 * D)])
    pltpu.sync_copy(zb, sh_sq.at[pl.ds(z0, 40 * D)])
    z16 = pl.multiple_of((40 * 16) * s, 8)
    pltpu.sync_copy(zb.at[pl.ds(0, 40 * 16)], sh_cnt.at[pl.ds(z16, 40 * 16)])
    for j in range(_NJ):
        sl = pl.ds(16 * j, 16)
        tot_sum[sl] = zvec
        tot_sq[sl] = zvec
        snap_sum[sl] = zvec
        snap_sq[sl] = zvec
    plsc.subcore_barrier()

    sb0, sb1, base_al = _worker_range(gid)
    nblk = sb1 - sb0
    pltpu.sync_copy(ids_hbm.at[pl.ds(base_al, IDS_LEN)], ids_v)

    def _feats_copy(b, slot):
        boff = pl.multiple_of((sb0 + b) * BLKW, 8)
        doff = pl.multiple_of(slot * BLKW, 8)
        return pltpu.make_async_copy(feats_hbm.at[pl.ds(boff, BLKW)],
                                     buf.at[pl.ds(doff, BLKW)], sem)

    _feats_copy(0, 0).start()

    def _flush(cur, cnt, loc, locsq):
        iota = _iota16()
        for j in range(_NJ):
            sl = pl.ds(16 * j, 16)
            t = tot_sum[sl] + loc[j]
            q = tot_sq[sl] + locsq[j]
            st_sum[sl] = t - snap_sum[sl]
            st_sq[sl] = q - snap_sq[sl]
            snap_sum[sl] = t
            snap_sq[sl] = q
            st_idx[sl] = iota + (cur * D + 16 * j)
        st_cnt[pl.ds(0, 16)] = jnp.full((16,), cnt, jnp.float32)
        st_idx16[pl.ds(0, 16)] = iota + cur * 16
        pltpu.sync_copy(st_sum, sh_sum.at[st_idx], add=True)
        pltpu.sync_copy(st_sq, sh_sq.at[st_idx], add=True)
        pltpu.sync_copy(st_cnt, sh_cnt.at[st_idx16], add=True)

    def _block(b, carry):
        _feats_copy(b, b % 2).wait()

        @pl.when(b + 1 < nblk)
        def _():
            _feats_copy(b + 1, (b + 1) % 2).start()

        bb = pl.multiple_of((b % 2) * BLKW, 8)

        def _group(g, carry):
            cur, srows = carry
            row0 = b * BLK + g * 16
            idv = ids_v[pl.ds(row0, 16)]
            first = idv[0]
            last = idv[15]
            uniform = first == last
            zloc = [zvec] * _NJ
            chg0 = uniform & (first != cur)

            @pl.when(chg0)
            def _():
                _flush(cur, (row0 - srows).astype(jnp.float32), zloc, zloc)

            cur = jnp.where(chg0, first, cur)
            srows = jnp.where(chg0, row0, srows)

            def _fast(cur, srows):
                loc = [zvec] * _NJ
                locsq = [zvec] * _NJ
                for r in range(16):
                    o = bb + (g * 16 + r) * D
                    for j in range(_NJ):
                        x = buf[pl.ds(o + 16 * j, 16)]
                        loc[j] = loc[j] + x
                        locsq[j] = locsq[j] + x * x
                for j in range(_NJ):
                    sl = pl.ds(16 * j, 16)
                    tot_sum[sl] = tot_sum[sl] + loc[j]
                    tot_sq[sl] = tot_sq[sl] + locsq[j]
                return cur, srows

            def _slow(cur, srows):
                def _srow(r, carry):
                    cur, srows, *vs = carry
                    loc, locsq = list(vs[:_NJ]), list(vs[_NJ:])
                    row = row0 + r
                    sid = ids_v[pl.ds(row, 16)][0]
                    chg = sid != cur

                    @pl.when(chg)
                    def _():
                        _flush(cur, (row - srows).astype(jnp.float32),
                               loc, locsq)

                    cur = jnp.where(chg, sid, cur)
                    srows = jnp.where(chg, row, srows)
                    o = bb + (g * 16 + r) * D
                    for j in range(_NJ):
                        x = buf[pl.ds(o + 16 * j, 16)]
                        loc[j] = loc[j] + x
                        locsq[j] = locsq[j] + x * x
                    return (cur, srows, *loc, *locsq)

                cur, srows, *vs = lax.fori_loop(
                    0, 16, _srow, (cur, srows, *([zvec] * (2 * _NJ))))
                for j in range(_NJ):
                    sl = pl.ds(16 * j, 16)
                    tot_sum[sl] = tot_sum[sl] + vs[j]
                    tot_sq[sl] = tot_sq[sl] + vs[_NJ + j]
                return cur, srows

            return lax.cond(uniform, _fast, _slow, cur, srows)

        return lax.fori_loop(0, NG, _group, carry)

    cur0 = ids_v[pl.ds(0, 16)][0]
    cur, srows = lax.fori_loop(0, nblk, _block, (cur0, jnp.int32(0)))
    zloc = [jnp.zeros((16,), jnp.float32)] * _NJ
    _flush(cur, (nblk * BLK - srows).astype(jnp.float32), zloc, zloc)

    plsc.subcore_barrier()
    p0 = pl.multiple_of((32 * D) * s, 8)
    o0 = pl.multiple_of(c * (NSEG * D) + (32 * D) * s, 8)
    pltpu.sync_copy(sh_sum.at[pl.ds(p0, 32 * D)],
                    psum_hbm.at[pl.ds(o0, 32 * D)])
    pltpu.sync_copy(sh_sq.at[pl.ds(p0, 32 * D)],
                    psq_hbm.at[pl.ds(o0, 32 * D)])
    p16 = pl.multiple_of((32 * 16) * s, 8)
    o16 = pl.multiple_of(c * (NSEG * 16) + (32 * 16) * s, 8)
    pltpu.sync_copy(sh_cnt.at[pl.ds(p16, 32 * 16)],
                    pcnt_hbm.at[pl.ds(o16, 32 * 16)])


def _tc_stats(psum, psq, pcnt, weight, bias, mean_scale):
    def body(psum_ref, psq_ref, pcnt_ref, w_ref, b_ref, ms_ref,
             a_ref, c_ref):
        sum_ = psum_ref[0] + psum_ref[1]
        sq = psq_ref[0] + psq_ref[1]
        cnt = pcnt_ref[0, :, 0:1] + pcnt_ref[1, :, 0:1]
        n = jnp.maximum(cnt, 1.0)
        m = sum_ / n
        msc = m * ms_ref[...]
        varsum = jnp.maximum(sq - msc * (2.0 * sum_ - n * msc), 0.0)
        std = jnp.sqrt(varsum / n + EPS)
        a = w_ref[...] / std
        a_ref[...] = a
        c_ref[...] = b_ref[...] - a * msc

    return pl.pallas_call(
        body,
        out_shape=(
            jax.ShapeDtypeStruct((NSEG, D), jnp.float32),
            jax.ShapeDtypeStruct((NSEG, D), jnp.float32),
        ),
    )(psum, psq, pcnt, weight, bias, mean_scale)


@functools.partial(
    pl.kernel,
    out_type=jax.ShapeDtypeStruct((N * D,), jnp.float32),
    mesh=_mesh,
    scratch_types=[
        pltpu.VMEM((IDS_LEN,), jnp.int32),
        pltpu.VMEM((2 * BLKW,), jnp.float32),
        pltpu.VMEM((2 * BLKW,), jnp.float32),
        pltpu.VMEM((D,), jnp.float32),
        pltpu.VMEM((D,), jnp.float32),
        pltpu.SemaphoreType.DMA,
        pltpu.SemaphoreType.DMA,
    ],
)
def _sc_normalize(feats_hbm, ids_hbm, a_hbm, c_hbm, out_hbm,
                  ids_v, buf, obuf, arow, crow, sem_in, sem_out):
    c = lax.axis_index("c")
    s = lax.axis_index("s")
    gid = c * NS + s
    sb0, sb1, base_al = _worker_range(gid)
    nblk = sb1 - sb0
    pltpu.sync_copy(ids_hbm.at[pl.ds(base_al, IDS_LEN)], ids_v)

    def _in_copy(b, slot):
        boff = pl.multiple_of((sb0 + b) * BLKW, 8)
        doff = pl.multiple_of(slot * BLKW, 8)
        return pltpu.make_async_copy(feats_hbm.at[pl.ds(boff, BLKW)],
                                     buf.at[pl.ds(doff, BLKW)], sem_in)

    def _out_copy(b, slot):
        boff = pl.multiple_of((sb0 + b) * BLKW, 8)
        soff = pl.multiple_of(slot * BLKW, 8)
        return pltpu.make_async_copy(obuf.at[pl.ds(soff, BLKW)],
                                     out_hbm.at[pl.ds(boff, BLKW)], sem_out)

    _in_copy(0, 0).start()

    def _pull(sid):
        soff = pl.multiple_of(sid * D, 8)
        pltpu.sync_copy(a_hbm.at[pl.ds(soff, D)], arow)
        pltpu.sync_copy(c_hbm.at[pl.ds(soff, D)], crow)

    def _block(b, cur):
        _in_copy(b, b % 2).wait()

        @pl.when(b + 1 < nblk)
        def _():
            _in_copy(b + 1, (b + 1) % 2).start()

        @pl.when(b >= 2)
        def _():
            _out_copy(b - 2, b % 2).wait()

        bb = pl.multiple_of((b % 2) * BLKW, 8)

        def _group(g, cur):
            row0 = b * BLK + g * 16
            idv = ids_v[pl.ds(row0, 16)]
            first = idv[0]
            last = idv[15]
            uniform = first == last

            @pl.when(uniform & (first != cur))
            def _():
                _pull(first)

            def _fast(cur):
                avs = [arow[pl.ds(16 * j, 16)] for j in range(D // 16)]
                cvs = [crow[pl.ds(16 * j, 16)] for j in range(D // 16)]

                for r in range(16):
                    o = bb + (g * 16 + r) * D
                    for j in range(D // 16):
                        x = buf[pl.ds(o + 16 * j, 16)]
                        obuf[pl.ds(o + 16 * j, 16)] = avs[j] * x + cvs[j]
                return last

            def _slow(cur):
                def _srow(r, cur):
                    sid = ids_v[pl.ds(row0 + r, 16)][0]

                    @pl.when(sid != cur)
                    def _():
                        _pull(sid)

                    o = bb + (g * 16 + r) * D
                    for j in range(D // 16):
                        x = buf[pl.ds(o + 16 * j, 16)]
                        obuf[pl.ds(o + 16 * j, 16)] = (
                            arow[pl.ds(16 * j, 16)] * x
                            + crow[pl.ds(16 * j, 16)])
                    return jnp.where(sid != cur, sid, cur)
                return lax.fori_loop(0, 16, _srow, cur)

            return lax.cond(uniform, _fast, _slow, cur)

        cur = lax.fori_loop(0, NG, _group, cur)
        _out_copy(b, b % 2).start()
        return cur

    lax.fori_loop(0, nblk, _block, jnp.int32(-1))

    @pl.when(nblk >= 2)
    def _():
        _out_copy(nblk - 2, nblk % 2).wait()
    _out_copy(nblk - 1, (nblk - 1) % 2).wait()


def kernel(feats, segment_ids, weight, bias, mean_scale):
    ids = segment_ids.astype(jnp.int32)
    ids_pad = jnp.pad(ids, (0, 128))
    feats_flat = feats.reshape(N * D)
    psum, psq, pcnt = _sc_accumulate(feats_flat, ids_pad)
    psum = psum.reshape(NC, NSEG, D)
    psq = psq.reshape(NC, NSEG, D)
    pcnt = pcnt.reshape(NC, NSEG, 16)
    a, c = _tc_stats(psum, psq, pcnt, weight.reshape(1, D),
                     bias.reshape(1, D), mean_scale.reshape(1, D))
    out = _sc_normalize(feats_flat, ids_pad, a.reshape(NSEG * D),
                        c.reshape(NSEG * D))
    return out.reshape(N, D)


# speedup vs baseline: 4.5023x; 1.0088x over previous
import functools

import jax
import jax.numpy as jnp
from jax import lax
from jax.experimental import pallas as pl
from jax.experimental.pallas import tpu as pltpu
from jax.experimental.pallas import tpu_sc as plsc

N = 100000
D = 128
NSEG = 512
EPS = 1e-05

NC = 2
NS = 16
NW = NC * NS
BLK = 160
NG = BLK // 16
SB = N // BLK
BLKW = BLK * D
IDS_LEN = 3216
ACC_ROWS = 640

_mesh = plsc.VectorSubcoreMesh(core_axis_name="c", subcore_axis_name="s")
_NJ = D // 16


def _iota16():
    return lax.iota(jnp.int32, 16)


def _worker_range(gid):
    sb0 = (SB * gid) // NW
    sb1 = (SB * (gid + 1)) // NW
    base_al = pl.multiple_of(sb0 * BLK, 8)
    return sb0, sb1, base_al


@functools.partial(
    pl.kernel,
    out_type=(
        jax.ShapeDtypeStruct((NC * NSEG * D,), jnp.float32),
        jax.ShapeDtypeStruct((NC * NSEG * D,), jnp.float32),
        jax.ShapeDtypeStruct((NC * NSEG * 16,), jnp.float32),
    ),
    mesh=_mesh,
    scratch_types=[
        pltpu.VMEM((IDS_LEN,), jnp.int32),
        pltpu.VMEM((2 * BLKW,), jnp.float32),
        pltpu.VMEM((D,), jnp.float32),
        pltpu.VMEM((D,), jnp.float32),
        pltpu.VMEM((D,), jnp.float32),
        pltpu.VMEM((D,), jnp.float32),
        pltpu.VMEM((D,), jnp.float32),
        pltpu.VMEM((D,), jnp.float32),
        pltpu.VMEM((16,), jnp.float32),
        pltpu.VMEM((D,), jnp.int32),
        pltpu.VMEM((16,), jnp.int32),
        pltpu.VMEM((40 * D,), jnp.float32),
        pltpu.SemaphoreType.DMA,
        pltpu.VMEM_SHARED((ACC_ROWS * D,), jnp.float32),
        pltpu.VMEM_SHARED((ACC_ROWS * D,), jnp.float32),
        pltpu.VMEM_SHARED((ACC_ROWS * 16,), jnp.float32),
    ],
)
def _sc_accumulate(feats_hbm, ids_hbm, psum_hbm, psq_hbm, pcnt_hbm,
                   ids_v, buf, tot_sum, tot_sq, snap_sum, snap_sq,
                   st_sum, st_sq, st_cnt, st_idx, st_idx16, zb, sem,
                   sh_sum, sh_sq, sh_cnt):
    c = lax.axis_index("c")
    s = lax.axis_index("s")
    gid = c * NS + s
    zvec = jnp.zeros((16,), jnp.float32)

    def _zchunk(i, _):
        zb[pl.ds(i * 16, 16)] = zvec
        return 0
    lax.fori_loop(0, 40 * D // 16, _zchunk, 0)
    z0 = pl.multiple_of((40 * D) * s, 8)
    pltpu.sync_copy(zb, sh_sum.at[pl.ds(z0, 40 * D)])
    pltpu.sync_copy(zb, sh_sq.at[pl.ds(z0, 40 * D)])
    z16 = pl.multiple_of((40 * 16) * s, 8)
    pltpu.sync_copy(zb.at[pl.ds(0, 40 * 16)], sh_cnt.at[pl.ds(z16, 40 * 16)])
    for j in range(_NJ):
        sl = pl.ds(16 * j, 16)
        tot_sum[sl] = zvec
        tot_sq[sl] = zvec
        snap_sum[sl] = zvec
        snap_sq[sl] = zvec
    plsc.subcore_barrier()

    sb0, sb1, base_al = _worker_range(gid)
    nblk = sb1 - sb0
    pltpu.sync_copy(ids_hbm.at[pl.ds(base_al, IDS_LEN)], ids_v)

    def _feats_copy(b, slot):
        boff = pl.multiple_of((sb0 + b) * BLKW, 8)
        doff = pl.multiple_of(slot * BLKW, 8)
        return pltpu.make_async_copy(feats_hbm.at[pl.ds(boff, BLKW)],
                                     buf.at[pl.ds(doff, BLKW)], sem)

    _feats_copy(0, 0).start()

    def _flush(cur, cnt, loc, locsq):
        iota = _iota16()
        for j in range(_NJ):
            sl = pl.ds(16 * j, 16)
            t = tot_sum[sl] + loc[j]
            q = tot_sq[sl] + locsq[j]
            st_sum[sl] = t - snap_sum[sl]
            st_sq[sl] = q - snap_sq[sl]
            snap_sum[sl] = t
            snap_sq[sl] = q
            st_idx[sl] = iota + (cur * D + 16 * j)
        st_cnt[pl.ds(0, 16)] = jnp.full((16,), cnt, jnp.float32)
        st_idx16[pl.ds(0, 16)] = iota + cur * 16
        pltpu.sync_copy(st_sum, sh_sum.at[st_idx], add=True)
        pltpu.sync_copy(st_sq, sh_sq.at[st_idx], add=True)
        pltpu.sync_copy(st_cnt, sh_cnt.at[st_idx16], add=True)

    def _block(b, carry):
        _feats_copy(b, b % 2).wait()

        @pl.when(b + 1 < nblk)
        def _():
            _feats_copy(b + 1, (b + 1) % 2).start()

        bb = pl.multiple_of((b % 2) * BLKW, 8)

        def _group(g, carry):
            cur, srows = carry
            row0 = b * BLK + g * 16
            idv = ids_v[pl.ds(row0, 16)]
            first = idv[0]
            last = idv[15]
            uniform = first == last
            zloc = [zvec] * _NJ
            chg0 = uniform & (first != cur)

            @pl.when(chg0)
            def _():
                _flush(cur, (row0 - srows).astype(jnp.float32), zloc, zloc)

            cur = jnp.where(chg0, first, cur)
            srows = jnp.where(chg0, row0, srows)

            def _fast(cur, srows):
                loc = [zvec] * _NJ
                locsq = [zvec] * _NJ
                for r in range(16):
                    o = bb + (g * 16 + r) * D
                    for j in range(_NJ):
                        x = buf[pl.ds(o + 16 * j, 16)]
                        loc[j] = loc[j] + x
                        locsq[j] = locsq[j] + x * x
                for j in range(_NJ):
                    sl = pl.ds(16 * j, 16)
                    tot_sum[sl] = tot_sum[sl] + loc[j]
                    tot_sq[sl] = tot_sq[sl] + locsq[j]
                return cur, srows

            def _slow(cur, srows):
                def _srow(r, carry):
                    cur, srows, *vs = carry
                    loc, locsq = list(vs[:_NJ]), list(vs[_NJ:])
                    row = row0 + r
                    sid = ids_v[pl.ds(row, 16)][0]
                    chg = sid != cur

                    @pl.when(chg)
                    def _():
                        _flush(cur, (row - srows).astype(jnp.float32),
                               loc, locsq)

                    cur = jnp.where(chg, sid, cur)
                    srows = jnp.where(chg, row, srows)
                    o = bb + (g * 16 + r) * D
                    for j in range(_NJ):
                        x = buf[pl.ds(o + 16 * j, 16)]
                        loc[j] = loc[j] + x
                        locsq[j] = locsq[j] + x * x
                    return (cur, srows, *loc, *locsq)

                cur, srows, *vs = lax.fori_loop(
                    0, 16, _srow, (cur, srows, *([zvec] * (2 * _NJ))))
                for j in range(_NJ):
                    sl = pl.ds(16 * j, 16)
                    tot_sum[sl] = tot_sum[sl] + vs[j]
                    tot_sq[sl] = tot_sq[sl] + vs[_NJ + j]
                return cur, srows

            return lax.cond(uniform, _fast, _slow, cur, srows)

        return lax.fori_loop(0, NG, _group, carry)

    cur0 = ids_v[pl.ds(0, 16)][0]
    cur, srows = lax.fori_loop(0, nblk, _block, (cur0, jnp.int32(0)))
    zloc = [jnp.zeros((16,), jnp.float32)] * _NJ
    _flush(cur, (nblk * BLK - srows).astype(jnp.float32), zloc, zloc)

    plsc.subcore_barrier()
    p0 = pl.multiple_of((32 * D) * s, 8)
    o0 = pl.multiple_of(c * (NSEG * D) + (32 * D) * s, 8)
    pltpu.sync_copy(sh_sum.at[pl.ds(p0, 32 * D)],
                    psum_hbm.at[pl.ds(o0, 32 * D)])
    pltpu.sync_copy(sh_sq.at[pl.ds(p0, 32 * D)],
                    psq_hbm.at[pl.ds(o0, 32 * D)])
    p16 = pl.multiple_of((32 * 16) * s, 8)
    o16 = pl.multiple_of(c * (NSEG * 16) + (32 * 16) * s, 8)
    pltpu.sync_copy(sh_cnt.at[pl.ds(p16, 32 * 16)],
                    pcnt_hbm.at[pl.ds(o16, 32 * 16)])


@functools.partial(
    pl.kernel,
    out_type=jax.ShapeDtypeStruct((N * D,), jnp.float32),
    mesh=_mesh,
    scratch_types=[
        pltpu.VMEM((IDS_LEN,), jnp.int32),
        pltpu.VMEM((2 * BLKW,), jnp.float32),
        pltpu.VMEM((2 * BLKW,), jnp.float32),
        pltpu.VMEM((D,), jnp.float32),
        pltpu.VMEM((D,), jnp.float32),
        pltpu.VMEM((3 * D,), jnp.float32),
        pltpu.VMEM((2 * 32 * D,), jnp.float32),
        pltpu.VMEM((2 * 32 * D,), jnp.float32),
        pltpu.VMEM((2 * 32 * 16,), jnp.float32),
        pltpu.VMEM((32 * D,), jnp.float32),
        pltpu.VMEM((32 * D,), jnp.float32),
        pltpu.SemaphoreType.DMA,
        pltpu.SemaphoreType.DMA,
        pltpu.VMEM_SHARED((NSEG * D,), jnp.float32),
        pltpu.VMEM_SHARED((NSEG * D,), jnp.float32),
    ],
)
def _sc_normalize(feats_hbm, ids_hbm, psum_hbm, psq_hbm, pcnt_hbm, wbm_hbm,
                  out_hbm, ids_v, buf, obuf, arow, crow, wbm, ps, qs, cs,
                  sa, sc_, sem_in, sem_out, sh_a, sh_c):
    c = lax.axis_index("c")
    s = lax.axis_index("s")
    gid = c * NS + s

    o0 = pl.multiple_of((32 * D) * s, 8)
    o1 = pl.multiple_of(NSEG * D + (32 * D) * s, 8)
    pltpu.sync_copy(psum_hbm.at[pl.ds(o0, 32 * D)], ps.at[pl.ds(0, 32 * D)])
    pltpu.sync_copy(psum_hbm.at[pl.ds(o1, 32 * D)],
                    ps.at[pl.ds(32 * D, 32 * D)])
    pltpu.sync_copy(psq_hbm.at[pl.ds(o0, 32 * D)], qs.at[pl.ds(0, 32 * D)])
    pltpu.sync_copy(psq_hbm.at[pl.ds(o1, 32 * D)],
                    qs.at[pl.ds(32 * D, 32 * D)])
    c0 = pl.multiple_of((32 * 16) * s, 8)
    c1 = pl.multiple_of(NSEG * 16 + (32 * 16) * s, 8)
    pltpu.sync_copy(pcnt_hbm.at[pl.ds(c0, 32 * 16)], cs.at[pl.ds(0, 32 * 16)])
    pltpu.sync_copy(pcnt_hbm.at[pl.ds(c1, 32 * 16)],
                    cs.at[pl.ds(32 * 16, 32 * 16)])
    pltpu.sync_copy(wbm_hbm, wbm)

    half = jnp.full((16,), 0.5, jnp.float32)
    three_half = jnp.full((16,), 1.5, jnp.float32)
    magic = jnp.full((16,), 0x5f3759df, jnp.int32)

    def _seg(k, _):
        cnt = cs[pl.ds(k * 16, 16)] + cs[pl.ds(32 * 16 + k * 16, 16)]
        n = jnp.maximum(cnt, 1.0)
        inv_n = 1.0 / n
        for j in range(_NJ):
            sl = pl.ds(k * D + 16 * j, 16)
            sl2 = pl.ds(32 * D + k * D + 16 * j, 16)
            w = wbm[pl.ds(16 * j, 16)]
            bia = wbm[pl.ds(D + 16 * j, 16)]
            msf = wbm[pl.ds(2 * D + 16 * j, 16)]
            su = ps[sl] + ps[sl2]
            sq = qs[sl] + qs[sl2]
            m = su * inv_n
            msc = m * msf
            varsum = jnp.maximum(sq - msc * (2.0 * su - n * msc), 0.0)
            x = varsum * inv_n + EPS
            xi = lax.bitcast_convert_type(x, jnp.int32)
            y = lax.bitcast_convert_type(magic - (xi >> 1), jnp.float32)
            hx = half * x
            for _i in range(3):
                y = y * (three_half - hx * y * y)
            a = w * y
            sa[pl.ds(k * D + 16 * j, 16)] = a
            sc_[pl.ds(k * D + 16 * j, 16)] = bia - a * msc
        return 0

    lax.fori_loop(0, 32, _seg, 0)
    t0 = pl.multiple_of((32 * D) * s, 8)
    pltpu.sync_copy(sa, sh_a.at[pl.ds(t0, 32 * D)])
    pltpu.sync_copy(sc_, sh_c.at[pl.ds(t0, 32 * D)])
    plsc.subcore_barrier()

    sb0, sb1, base_al = _worker_range(gid)
    nblk = sb1 - sb0
    pltpu.sync_copy(ids_hbm.at[pl.ds(base_al, IDS_LEN)], ids_v)

    def _in_copy(b, slot):
        boff = pl.multiple_of((sb0 + b) * BLKW, 8)
        doff = pl.multiple_of(slot * BLKW, 8)
        return pltpu.make_async_copy(feats_hbm.at[pl.ds(boff, BLKW)],
                                     buf.at[pl.ds(doff, BLKW)], sem_in)

    def _out_copy(b, slot):
        boff = pl.multiple_of((sb0 + b) * BLKW, 8)
        soff = pl.multiple_of(slot * BLKW, 8)
        return pltpu.make_async_copy(obuf.at[pl.ds(soff, BLKW)],
                                     out_hbm.at[pl.ds(boff, BLKW)], sem_out)

    _in_copy(0, 0).start()

    def _pull(sid):
        soff = pl.multiple_of(sid * D, 8)
        pltpu.sync_copy(sh_a.at[pl.ds(soff, D)], arow)
        pltpu.sync_copy(sh_c.at[pl.ds(soff, D)], crow)

    def _block(b, cur):
        _in_copy(b, b % 2).wait()

        @pl.when(b + 1 < nblk)
        def _():
            _in_copy(b + 1, (b + 1) % 2).start()

        @pl.when(b >= 2)
        def _():
            _out_copy(b - 2, b % 2).wait()

        bb = pl.multiple_of((b % 2) * BLKW, 8)

        def _group(g, cur):
            row0 = b * BLK + g * 16
            idv = ids_v[pl.ds(row0, 16)]
            first = idv[0]
            last = idv[15]
            uniform = first == last

            @pl.when(uniform & (first != cur))
            def _():
                _pull(first)

            def _fast(cur):
                avs = [arow[pl.ds(16 * j, 16)] for j in range(_NJ)]
                cvs = [crow[pl.ds(16 * j, 16)] for j in range(_NJ)]
                for r in range(16):
                    o = bb + (g * 16 + r) * D
                    for j in range(_NJ):
                        x = buf[pl.ds(o + 16 * j, 16)]
                        obuf[pl.ds(o + 16 * j, 16)] = avs[j] * x + cvs[j]
                return last

            def _slow(cur):
                def _srow(r, cur):
                    sid = ids_v[pl.ds(row0 + r, 16)][0]

                    @pl.when(sid != cur)
                    def _():
                        _pull(sid)

                    o = bb + (g * 16 + r) * D
                    for j in range(_NJ):
                        x = buf[pl.ds(o + 16 * j, 16)]
                        obuf[pl.ds(o + 16 * j, 16)] = (
                            arow[pl.ds(16 * j, 16)] * x
                            + crow[pl.ds(16 * j, 16)])
                    return jnp.where(sid != cur, sid, cur)
                return lax.fori_loop(0, 16, _srow, cur)

            return lax.cond(uniform, _fast, _slow, cur)

        cur = lax.fori_loop(0, NG, _group, cur)
        _out_copy(b, b % 2).start()
        return cur

    lax.fori_loop(0, nblk, _block, jnp.int32(-1))

    @pl.when(nblk >= 2)
    def _():
        _out_copy(nblk - 2, nblk % 2).wait()
    _out_copy(nblk - 1, (nblk - 1) % 2).wait()


def kernel(feats, segment_ids, weight, bias, mean_scale):
    ids = segment_ids.astype(jnp.int32)
    ids_pad = jnp.pad(ids, (0, 128))
    feats_flat = feats.reshape(N * D)
    psum, psq, pcnt = _sc_accumulate(feats_flat, ids_pad)
    wbm = jnp.concatenate([weight, bias, mean_scale])
    out = _sc_normalize(feats_flat, ids_pad, psum, psq, pcnt, wbm)
    return out.reshape(N, D)


# speedup vs baseline: 8.4598x; 1.8790x over previous
import functools

import jax
import jax.numpy as jnp
from jax import lax
from jax.experimental import pallas as pl
from jax.experimental.pallas import tpu as pltpu
from jax.experimental.pallas import tpu_sc as plsc

N = 100000
D = 128
NSEG = 512
EPS = 1e-05

NC = 2
NS = 16
NW = NC * NS
BLK = 160
NG = BLK // 16
SB = N // BLK
BLKW = BLK * D
IDS_LEN = 3216
ACC_ROWS = 640

_mesh = plsc.VectorSubcoreMesh(core_axis_name="c", subcore_axis_name="s")
_NJ = D // 16


def _iota16():
    return lax.iota(jnp.int32, 16)


def _worker_range(gid):
    sb0 = (SB * gid) // NW
    sb1 = (SB * (gid + 1)) // NW
    base_al = pl.multiple_of(sb0 * BLK, 8)
    return sb0, sb1, base_al


@functools.partial(
    pl.kernel,
    out_type=(
        jax.ShapeDtypeStruct((NC * NSEG * D,), jnp.float32),
        jax.ShapeDtypeStruct((NC * NSEG * D,), jnp.float32),
        jax.ShapeDtypeStruct((NC * NSEG * 16,), jnp.float32),
    ),
    mesh=_mesh,
    scratch_types=[
        pltpu.VMEM((IDS_LEN,), jnp.int32),
        pltpu.VMEM((2 * BLKW,), jnp.float32),
        pltpu.VMEM((D,), jnp.float32),
        pltpu.VMEM((D,), jnp.float32),
        pltpu.VMEM((D,), jnp.float32),
        pltpu.VMEM((D,), jnp.float32),
        pltpu.VMEM((D,), jnp.float32),
        pltpu.VMEM((D,), jnp.float32),
        pltpu.VMEM((16,), jnp.float32),
        pltpu.VMEM((D,), jnp.int32),
        pltpu.VMEM((16,), jnp.int32),
        pltpu.VMEM((40 * D,), jnp.float32),
        pltpu.SemaphoreType.DMA,
        pltpu.VMEM_SHARED((ACC_ROWS * D,), jnp.float32),
        pltpu.VMEM_SHARED((ACC_ROWS * D,), jnp.float32),
        pltpu.VMEM_SHARED((ACC_ROWS * 16,), jnp.float32),
    ],
)
def _sc_accumulate(feats_hbm, ids_hbm, psum_hbm, psq_hbm, pcnt_hbm,
                   ids_v, buf, tot_sum, tot_sq, snap_sum, snap_sq,
                   st_sum, st_sq, st_cnt, st_idx, st_idx16, zb, sem,
                   sh_sum, sh_sq, sh_cnt):
    c = lax.axis_index("c")
    s = lax.axis_index("s")
    gid = c * NS + s
    zvec = jnp.zeros((16,), jnp.float32)

    def _zchunk(i, _):
        zb[pl.ds(i * 16, 16)] = zvec
        return 0
    lax.fori_loop(0, 40 * D // 16, _zchunk, 0)
    z0 = pl.multiple_of((40 * D) * s, 8)
    pltpu.sync_copy(zb, sh_sum.at[pl.ds(z0, 40 * D)])
    pltpu.sync_copy(zb, sh_sq.at[pl.ds(z0, 40 * D)])
    z16 = pl.multiple_of((40 * 16) * s, 8)
    pltpu.sync_copy(zb.at[pl.ds(0, 40 * 16)], sh_cnt.at[pl.ds(z16, 40 * 16)])
    for j in range(_NJ):
        sl = pl.ds(16 * j, 16)
        tot_sum[sl] = zvec
        tot_sq[sl] = zvec
        snap_sum[sl] = zvec
        snap_sq[sl] = zvec
    plsc.subcore_barrier()

    sb0, sb1, base_al = _worker_range(gid)
    nblk = sb1 - sb0
    pltpu.sync_copy(ids_hbm.at[pl.ds(base_al, IDS_LEN)], ids_v)

    def _feats_copy(b, slot):
        boff = pl.multiple_of((sb0 + b) * BLKW, 8)
        doff = pl.multiple_of(slot * BLKW, 8)
        return pltpu.make_async_copy(feats_hbm.at[pl.ds(boff, BLKW)],
                                     buf.at[pl.ds(doff, BLKW)], sem)

    _feats_copy(0, 0).start()

    def _flush(cur, cnt, loc, locsq):
        iota = _iota16()
        for j in range(_NJ):
            sl = pl.ds(16 * j, 16)
            t = tot_sum[sl] + loc[j]
            q = tot_sq[sl] + locsq[j]
            st_sum[sl] = t - snap_sum[sl]
            st_sq[sl] = q - snap_sq[sl]
            snap_sum[sl] = t
            snap_sq[sl] = q
            st_idx[sl] = iota + (cur * D + 16 * j)
        st_cnt[pl.ds(0, 16)] = jnp.full((16,), cnt, jnp.float32)
        st_idx16[pl.ds(0, 16)] = iota + cur * 16
        pltpu.sync_copy(st_sum, sh_sum.at[st_idx], add=True)
        pltpu.sync_copy(st_sq, sh_sq.at[st_idx], add=True)
        pltpu.sync_copy(st_cnt, sh_cnt.at[st_idx16], add=True)

    def _block(b, carry):
        _feats_copy(b, b % 2).wait()

        @pl.when(b + 1 < nblk)
        def _():
            _feats_copy(b + 1, (b + 1) % 2).start()

        bb = pl.multiple_of((b % 2) * BLKW, 8)

        def _group(g, carry):
            cur, srows = carry
            row0 = b * BLK + g * 16
            idv = ids_v[pl.ds(row0, 16)]
            first = idv[0]
            last = idv[15]
            uniform = first == last
            zloc = [zvec] * _NJ
            chg0 = uniform & (first != cur)

            @pl.when(chg0)
            def _():
                _flush(cur, (row0 - srows).astype(jnp.float32), zloc, zloc)

            cur = jnp.where(chg0, first, cur)
            srows = jnp.where(chg0, row0, srows)

            def _fast(cur, srows):
                def _rows(r, cr):
                    loc, locsq = cr
                    loc, locsq = list(loc), list(locsq)
                    o = bb + (g * 16 + r) * D
                    for j in range(_NJ):
                        x = buf[pl.ds(o + 16 * j, 16)]
                        loc[j] = loc[j] + x
                        locsq[j] = locsq[j] + x * x
                    return (tuple(loc), tuple(locsq))

                loc, locsq = plsc.parallel_loop(
                    0, 16, step=1, unroll=4,
                    carry=(tuple([zvec] * _NJ), tuple([zvec] * _NJ)))(_rows)
                for j in range(_NJ):
                    sl = pl.ds(16 * j, 16)
                    tot_sum[sl] = tot_sum[sl] + loc[j]
                    tot_sq[sl] = tot_sq[sl] + locsq[j]
                return cur, srows

            def _slow(cur, srows):
                def _srow(r, carry):
                    cur, srows, *vs = carry
                    loc, locsq = list(vs[:_NJ]), list(vs[_NJ:])
                    row = row0 + r
                    sid = ids_v[pl.ds(row, 16)][0]
                    chg = sid != cur

                    @pl.when(chg)
                    def _():
                        _flush(cur, (row - srows).astype(jnp.float32),
                               loc, locsq)

                    cur = jnp.where(chg, sid, cur)
                    srows = jnp.where(chg, row, srows)
                    o = bb + (g * 16 + r) * D
                    for j in range(_NJ):
                        x = buf[pl.ds(o + 16 * j, 16)]
                        loc[j] = loc[j] + x
                        locsq[j] = locsq[j] + x * x
                    return (cur, srows, *loc, *locsq)

                cur, srows, *vs = lax.fori_loop(
                    0, 16, _srow, (cur, srows, *([zvec] * (2 * _NJ))))
                for j in range(_NJ):
                    sl = pl.ds(16 * j, 16)
                    tot_sum[sl] = tot_sum[sl] + vs[j]
                    tot_sq[sl] = tot_sq[sl] + vs[_NJ + j]
                return cur, srows

            return lax.cond(uniform, _fast, _slow, cur, srows)

        return lax.fori_loop(0, NG, _group, carry)

    cur0 = ids_v[pl.ds(0, 16)][0]
    cur, srows = lax.fori_loop(0, nblk, _block, (cur0, jnp.int32(0)))
    zloc = [jnp.zeros((16,), jnp.float32)] * _NJ
    _flush(cur, (nblk * BLK - srows).astype(jnp.float32), zloc, zloc)

    plsc.subcore_barrier()
    p0 = pl.multiple_of((32 * D) * s, 8)
    o0 = pl.multiple_of(c * (NSEG * D) + (32 * D) * s, 8)
    pltpu.sync_copy(sh_sum.at[pl.ds(p0, 32 * D)],
                    psum_hbm.at[pl.ds(o0, 32 * D)])
    pltpu.sync_copy(sh_sq.at[pl.ds(p0, 32 * D)],
                    psq_hbm.at[pl.ds(o0, 32 * D)])
    p16 = pl.multiple_of((32 * 16) * s, 8)
    o16 = pl.multiple_of(c * (NSEG * 16) + (32 * 16) * s, 8)
    pltpu.sync_copy(sh_cnt.at[pl.ds(p16, 32 * 16)],
                    pcnt_hbm.at[pl.ds(o16, 32 * 16)])


@functools.partial(
    pl.kernel,
    out_type=jax.ShapeDtypeStruct((N * D,), jnp.float32),
    mesh=_mesh,
    scratch_types=[
        pltpu.VMEM((IDS_LEN,), jnp.int32),
        pltpu.VMEM((2 * BLKW,), jnp.float32),
        pltpu.VMEM((2 * BLKW,), jnp.float32),
        pltpu.VMEM((D,), jnp.float32),
        pltpu.VMEM((D,), jnp.float32),
        pltpu.VMEM((3 * D,), jnp.float32),
        pltpu.VMEM((2 * 32 * D,), jnp.float32),
        pltpu.VMEM((2 * 32 * D,), jnp.float32),
        pltpu.VMEM((2 * 32 * 16,), jnp.float32),
        pltpu.VMEM((32 * D,), jnp.float32),
        pltpu.VMEM((32 * D,), jnp.float32),
        pltpu.SemaphoreType.DMA,
        pltpu.SemaphoreType.DMA,
        pltpu.VMEM_SHARED((NSEG * D,), jnp.float32),
        pltpu.VMEM_SHARED((NSEG * D,), jnp.float32),
    ],
)
def _sc_normalize(feats_hbm, ids_hbm, psum_hbm, psq_hbm, pcnt_hbm, wbm_hbm,
                  out_hbm, ids_v, buf, obuf, arow, crow, wbm, ps, qs, cs,
                  sa, sc_, sem_in, sem_out, sh_a, sh_c):
    c = lax.axis_index("c")
    s = lax.axis_index("s")
    gid = c * NS + s

    o0 = pl.multiple_of((32 * D) * s, 8)
    o1 = pl.multiple_of(NSEG * D + (32 * D) * s, 8)
    pltpu.sync_copy(psum_hbm.at[pl.ds(o0, 32 * D)], ps.at[pl.ds(0, 32 * D)])
    pltpu.sync_copy(psum_hbm.at[pl.ds(o1, 32 * D)],
                    ps.at[pl.ds(32 * D, 32 * D)])
    pltpu.sync_copy(psq_hbm.at[pl.ds(o0, 32 * D)], qs.at[pl.ds(0, 32 * D)])
    pltpu.sync_copy(psq_hbm.at[pl.ds(o1, 32 * D)],
                    qs.at[pl.ds(32 * D, 32 * D)])
    c0 = pl.multiple_of((32 * 16) * s, 8)
    c1 = pl.multiple_of(NSEG * 16 + (32 * 16) * s, 8)
    pltpu.sync_copy(pcnt_hbm.at[pl.ds(c0, 32 * 16)], cs.at[pl.ds(0, 32 * 16)])
    pltpu.sync_copy(pcnt_hbm.at[pl.ds(c1, 32 * 16)],
                    cs.at[pl.ds(32 * 16, 32 * 16)])
    pltpu.sync_copy(wbm_hbm, wbm)

    half = jnp.full((16,), 0.5, jnp.float32)
    three_half = jnp.full((16,), 1.5, jnp.float32)
    magic = jnp.full((16,), 0x5f3759df, jnp.int32)

    def _seg(k, _):
        cnt = cs[pl.ds(k * 16, 16)] + cs[pl.ds(32 * 16 + k * 16, 16)]
        n = jnp.maximum(cnt, 1.0)
        inv_n = 1.0 / n
        for j in range(_NJ):
            sl = pl.ds(k * D + 16 * j, 16)
            sl2 = pl.ds(32 * D + k * D + 16 * j, 16)
            w = wbm[pl.ds(16 * j, 16)]
            bia = wbm[pl.ds(D + 16 * j, 16)]
            msf = wbm[pl.ds(2 * D + 16 * j, 16)]
            su = ps[sl] + ps[sl2]
            sq = qs[sl] + qs[sl2]
            m = su * inv_n
            msc = m * msf
            varsum = jnp.maximum(sq - msc * (2.0 * su - n * msc), 0.0)
            x = varsum * inv_n + EPS
            xi = lax.bitcast_convert_type(x, jnp.int32)
            y = lax.bitcast_convert_type(magic - (xi >> 1), jnp.float32)
            hx = half * x
            for _i in range(3):
                y = y * (three_half - hx * y * y)
            a = w * y
            sa[pl.ds(k * D + 16 * j, 16)] = a
            sc_[pl.ds(k * D + 16 * j, 16)] = bia - a * msc
        return 0

    lax.fori_loop(0, 32, _seg, 0)
    t0 = pl.multiple_of((32 * D) * s, 8)
    pltpu.sync_copy(sa, sh_a.at[pl.ds(t0, 32 * D)])
    pltpu.sync_copy(sc_, sh_c.at[pl.ds(t0, 32 * D)])
    plsc.subcore_barrier()

    sb0, sb1, base_al = _worker_range(gid)
    nblk = sb1 - sb0
    pltpu.sync_copy(ids_hbm.at[pl.ds(base_al, IDS_LEN)], ids_v)

    def _in_copy(b, slot):
        boff = pl.multiple_of((sb0 + b) * BLKW, 8)
        doff = pl.multiple_of(slot * BLKW, 8)
        return pltpu.make_async_copy(feats_hbm.at[pl.ds(boff, BLKW)],
                                     buf.at[pl.ds(doff, BLKW)], sem_in)

    def _out_copy(b, slot):
        boff = pl.multiple_of((sb0 + b) * BLKW, 8)
        soff = pl.multiple_of(slot * BLKW, 8)
        return pltpu.make_async_copy(obuf.at[pl.ds(soff, BLKW)],
                                     out_hbm.at[pl.ds(boff, BLKW)], sem_out)

    _in_copy(0, 0).start()

    def _pull(sid):
        soff = pl.multiple_of(sid * D, 8)
        pltpu.sync_copy(sh_a.at[pl.ds(soff, D)], arow)
        pltpu.sync_copy(sh_c.at[pl.ds(soff, D)], crow)

    def _block(b, cur):
        _in_copy(b, b % 2).wait()

        @pl.when(b + 1 < nblk)
        def _():
            _in_copy(b + 1, (b + 1) % 2).start()

        @pl.when(b >= 2)
        def _():
            _out_copy(b - 2, b % 2).wait()

        bb = pl.multiple_of((b % 2) * BLKW, 8)

        def _group(g, cur):
            row0 = b * BLK + g * 16
            idv = ids_v[pl.ds(row0, 16)]
            first = idv[0]
            last = idv[15]
            uniform = first == last

            @pl.when(uniform & (first != cur))
            def _():
                _pull(first)

            def _fast(cur):
                avs = [arow[pl.ds(16 * j, 16)] for j in range(_NJ)]
                cvs = [crow[pl.ds(16 * j, 16)] for j in range(_NJ)]

                @plsc.parallel_loop(0, 16, step=1, unroll=4)
                def _rows(r):
                    o = bb + (g * 16 + r) * D
                    for j in range(_NJ):
                        x = buf[pl.ds(o + 16 * j, 16)]
                        obuf[pl.ds(o + 16 * j, 16)] = avs[j] * x + cvs[j]
                return last

            def _slow(cur):
                def _srow(r, cur):
                    sid = ids_v[pl.ds(row0 + r, 16)][0]

                    @pl.when(sid != cur)
                    def _():
                        _pull(sid)

                    o = bb + (g * 16 + r) * D
                    for j in range(_NJ):
                        x = buf[pl.ds(o + 16 * j, 16)]
                        obuf[pl.ds(o + 16 * j, 16)] = (
                            arow[pl.ds(16 * j, 16)] * x
                            + crow[pl.ds(16 * j, 16)])
                    return jnp.where(sid != cur, sid, cur)
                return lax.fori_loop(0, 16, _srow, cur)

            return lax.cond(uniform, _fast, _slow, cur)

        cur = lax.fori_loop(0, NG, _group, cur)
        _out_copy(b, b % 2).start()
        return cur

    lax.fori_loop(0, nblk, _block, jnp.int32(-1))

    @pl.when(nblk >= 2)
    def _():
        _out_copy(nblk - 2, nblk % 2).wait()
    _out_copy(nblk - 1, (nblk - 1) % 2).wait()


def kernel(feats, segment_ids, weight, bias, mean_scale):
    ids = segment_ids.astype(jnp.int32)
    ids_pad = jnp.pad(ids, (0, 128))
    feats_flat = feats.reshape(N * D)
    psum, psq, pcnt = _sc_accumulate(feats_flat, ids_pad)
    wbm = jnp.concatenate([weight, bias, mean_scale])
    out = _sc_normalize(feats_flat, ids_pad, psum, psq, pcnt, wbm)
    return out.reshape(N, D)


# speedup vs baseline: 8.6918x; 1.0274x over previous
import functools

import jax
import jax.numpy as jnp
from jax import lax
from jax.experimental import pallas as pl
from jax.experimental.pallas import tpu as pltpu
from jax.experimental.pallas import tpu_sc as plsc

N = 100000
D = 128
NSEG = 512
EPS = 1e-05

NC = 2
NS = 16
NW = NC * NS
BLK = 160
NG = BLK // 16
SB = N // BLK
BLKW = BLK * D
IDS_LEN = 3216
ACC_ROWS = 640

_mesh = plsc.VectorSubcoreMesh(core_axis_name="c", subcore_axis_name="s")
_NJ = D // 16


def _iota16():
    return lax.iota(jnp.int32, 16)


def _worker_range(gid):
    sb0 = (SB * gid) // NW
    sb1 = (SB * (gid + 1)) // NW
    base_al = pl.multiple_of(sb0 * BLK, 8)
    return sb0, sb1, base_al


@functools.partial(
    pl.kernel,
    out_type=(
        jax.ShapeDtypeStruct((NC * NSEG * D,), jnp.float32),
        jax.ShapeDtypeStruct((NC * NSEG * D,), jnp.float32),
        jax.ShapeDtypeStruct((NC * NSEG * 16,), jnp.float32),
    ),
    mesh=_mesh,
    scratch_types=[
        pltpu.VMEM((IDS_LEN,), jnp.int32),
        pltpu.VMEM((2 * BLKW,), jnp.float32),
        pltpu.VMEM((D,), jnp.float32),
        pltpu.VMEM((D,), jnp.float32),
        pltpu.VMEM((D,), jnp.float32),
        pltpu.VMEM((D,), jnp.float32),
        pltpu.VMEM((D,), jnp.float32),
        pltpu.VMEM((D,), jnp.float32),
        pltpu.VMEM((16,), jnp.float32),
        pltpu.VMEM((D,), jnp.int32),
        pltpu.VMEM((16,), jnp.int32),
        pltpu.VMEM((40 * D,), jnp.float32),
        pltpu.SemaphoreType.DMA,
        pltpu.VMEM_SHARED((ACC_ROWS * D,), jnp.float32),
        pltpu.VMEM_SHARED((ACC_ROWS * D,), jnp.float32),
        pltpu.VMEM_SHARED((ACC_ROWS * 16,), jnp.float32),
    ],
)
def _sc_accumulate(feats_hbm, ids_hbm, psum_hbm, psq_hbm, pcnt_hbm,
                   ids_v, buf, tot_sum, tot_sq, snap_sum, snap_sq,
                   st_sum, st_sq, st_cnt, st_idx, st_idx16, zb, sem,
                   sh_sum, sh_sq, sh_cnt):
    c = lax.axis_index("c")
    s = lax.axis_index("s")
    gid = c * NS + s
    zvec = jnp.zeros((16,), jnp.float32)

    def _zchunk(i, _):
        zb[pl.ds(i * 16, 16)] = zvec
        return 0
    lax.fori_loop(0, 40 * D // 16, _zchunk, 0)
    z0 = pl.multiple_of((40 * D) * s, 8)
    pltpu.sync_copy(zb, sh_sum.at[pl.ds(z0, 40 * D)])
    pltpu.sync_copy(zb, sh_sq.at[pl.ds(z0, 40 * D)])
    z16 = pl.multiple_of((40 * 16) * s, 8)
    pltpu.sync_copy(zb.at[pl.ds(0, 40 * 16)], sh_cnt.at[pl.ds(z16, 40 * 16)])
    for j in range(_NJ):
        sl = pl.ds(16 * j, 16)
        tot_sum[sl] = zvec
        tot_sq[sl] = zvec
        snap_sum[sl] = zvec
        snap_sq[sl] = zvec
    plsc.subcore_barrier()

    sb0, sb1, base_al = _worker_range(gid)
    nblk = sb1 - sb0
    pltpu.sync_copy(ids_hbm.at[pl.ds(base_al, IDS_LEN)], ids_v)

    def _feats_copy(b, slot):
        boff = pl.multiple_of((sb0 + b) * BLKW, 8)
        doff = pl.multiple_of(slot * BLKW, 8)
        return pltpu.make_async_copy(feats_hbm.at[pl.ds(boff, BLKW)],
                                     buf.at[pl.ds(doff, BLKW)], sem)

    _feats_copy(0, 0).start()

    def _flush(cur, cnt, loc, locsq):
        iota = _iota16()
        for j in range(_NJ):
            sl = pl.ds(16 * j, 16)
            t = tot_sum[sl] + loc[j]
            q = tot_sq[sl] + locsq[j]
            st_sum[sl] = t - snap_sum[sl]
            st_sq[sl] = q - snap_sq[sl]
            snap_sum[sl] = t
            snap_sq[sl] = q
            st_idx[sl] = iota + (cur * D + 16 * j)
        st_cnt[pl.ds(0, 16)] = jnp.full((16,), cnt, jnp.float32)
        st_idx16[pl.ds(0, 16)] = iota + cur * 16
        pltpu.sync_copy(st_sum, sh_sum.at[st_idx], add=True)
        pltpu.sync_copy(st_sq, sh_sq.at[st_idx], add=True)
        pltpu.sync_copy(st_cnt, sh_cnt.at[st_idx16], add=True)

    def _block(b, carry):
        _feats_copy(b, b % 2).wait()

        @pl.when(b + 1 < nblk)
        def _():
            _feats_copy(b + 1, (b + 1) % 2).start()

        bb = pl.multiple_of((b % 2) * BLKW, 8)

        def _group(g, carry):
            cur, srows = carry
            row0 = b * BLK + g * 16
            idv = ids_v[pl.ds(row0, 16)]
            first = idv[0]
            last = idv[15]
            uniform = first == last
            zloc = [zvec] * _NJ
            chg0 = uniform & (first != cur)

            @pl.when(chg0)
            def _():
                _flush(cur, (row0 - srows).astype(jnp.float32), zloc, zloc)

            cur = jnp.where(chg0, first, cur)
            srows = jnp.where(chg0, row0, srows)

            def _fast(cur, srows):
                def _rows(r, cr):
                    loc, locsq = cr
                    loc, locsq = list(loc), list(locsq)
                    o = bb + (g * 16 + r) * D
                    for j in range(_NJ):
                        x = buf[pl.ds(o + 16 * j, 16)]
                        loc[j] = loc[j] + x
                        locsq[j] = locsq[j] + x * x
                    return (tuple(loc), tuple(locsq))

                loc, locsq = plsc.parallel_loop(
                    0, 16, step=1, unroll=4,
                    carry=(tuple([zvec] * _NJ), tuple([zvec] * _NJ)))(_rows)
                for j in range(_NJ):
                    sl = pl.ds(16 * j, 16)
                    tot_sum[sl] = tot_sum[sl] + loc[j]
                    tot_sq[sl] = tot_sq[sl] + locsq[j]
                return cur, srows

            def _slow(cur, srows):
                def _srow(r, carry):
                    cur, srows, *vs = carry
                    loc, locsq = list(vs[:_NJ]), list(vs[_NJ:])
                    row = row0 + r
                    sid = ids_v[pl.ds(row, 16)][0]
                    chg = sid != cur

                    @pl.when(chg)
                    def _():
                        _flush(cur, (row - srows).astype(jnp.float32),
                               loc, locsq)

                    cur = jnp.where(chg, sid, cur)
                    srows = jnp.where(chg, row, srows)
                    o = bb + (g * 16 + r) * D
                    for j in range(_NJ):
                        x = buf[pl.ds(o + 16 * j, 16)]
                        loc[j] = loc[j] + x
                        locsq[j] = locsq[j] + x * x
                    return (cur, srows, *loc, *locsq)

                cur, srows, *vs = lax.fori_loop(
                    0, 16, _srow, (cur, srows, *([zvec] * (2 * _NJ))))
                for j in range(_NJ):
                    sl = pl.ds(16 * j, 16)
                    tot_sum[sl] = tot_sum[sl] + vs[j]
                    tot_sq[sl] = tot_sq[sl] + vs[_NJ + j]
                return cur, srows

            return lax.cond(uniform, _fast, _slow, cur, srows)

        return lax.fori_loop(0, NG, _group, carry)

    cur0 = ids_v[pl.ds(0, 16)][0]
    cur, srows = lax.fori_loop(0, nblk, _block, (cur0, jnp.int32(0)))
    zloc = [jnp.zeros((16,), jnp.float32)] * _NJ
    _flush(cur, (nblk * BLK - srows).astype(jnp.float32), zloc, zloc)

    plsc.subcore_barrier()
    p0 = pl.multiple_of((32 * D) * s, 8)
    o0 = pl.multiple_of(c * (NSEG * D) + (32 * D) * s, 8)
    pltpu.sync_copy(sh_sum.at[pl.ds(p0, 32 * D)],
                    psum_hbm.at[pl.ds(o0, 32 * D)])
    pltpu.sync_copy(sh_sq.at[pl.ds(p0, 32 * D)],
                    psq_hbm.at[pl.ds(o0, 32 * D)])
    p16 = pl.multiple_of((32 * 16) * s, 8)
    o16 = pl.multiple_of(c * (NSEG * 16) + (32 * 16) * s, 8)
    pltpu.sync_copy(sh_cnt.at[pl.ds(p16, 32 * 16)],
                    pcnt_hbm.at[pl.ds(o16, 32 * 16)])


@functools.partial(
    pl.kernel,
    out_type=jax.ShapeDtypeStruct((N * D,), jnp.float32),
    mesh=_mesh,
    scratch_types=[
        pltpu.VMEM((IDS_LEN,), jnp.int32),
        pltpu.VMEM((2 * BLKW,), jnp.float32),
        pltpu.VMEM((2 * BLKW,), jnp.float32),
        pltpu.VMEM((D,), jnp.float32),
        pltpu.VMEM((D,), jnp.float32),
        pltpu.VMEM((3 * D,), jnp.float32),
        pltpu.VMEM((2 * 32 * D,), jnp.float32),
        pltpu.VMEM((2 * 32 * D,), jnp.float32),
        pltpu.VMEM((2 * 32 * 16,), jnp.float32),
        pltpu.VMEM((32 * D,), jnp.float32),
        pltpu.VMEM((32 * D,), jnp.float32),
        pltpu.SemaphoreType.DMA,
        pltpu.SemaphoreType.DMA,
        pltpu.VMEM_SHARED((NSEG * D,), jnp.float32),
        pltpu.VMEM_SHARED((NSEG * D,), jnp.float32),
    ],
)
def _sc_normalize(feats_hbm, ids_hbm, psum_hbm, psq_hbm, pcnt_hbm, wbm_hbm,
                  out_hbm, ids_v, buf, obuf, arow, crow, wbm, ps, qs, cs,
                  sa, sc_, sem_in, sem_out, sh_a, sh_c):
    c = lax.axis_index("c")
    s = lax.axis_index("s")
    gid = c * NS + s

    o0 = pl.multiple_of((32 * D) * s, 8)
    o1 = pl.multiple_of(NSEG * D + (32 * D) * s, 8)
    pltpu.sync_copy(psum_hbm.at[pl.ds(o0, 32 * D)], ps.at[pl.ds(0, 32 * D)])
    pltpu.sync_copy(psum_hbm.at[pl.ds(o1, 32 * D)],
                    ps.at[pl.ds(32 * D, 32 * D)])
    pltpu.sync_copy(psq_hbm.at[pl.ds(o0, 32 * D)], qs.at[pl.ds(0, 32 * D)])
    pltpu.sync_copy(psq_hbm.at[pl.ds(o1, 32 * D)],
                    qs.at[pl.ds(32 * D, 32 * D)])
    c0 = pl.multiple_of((32 * 16) * s, 8)
    c1 = pl.multiple_of(NSEG * 16 + (32 * 16) * s, 8)
    pltpu.sync_copy(pcnt_hbm.at[pl.ds(c0, 32 * 16)], cs.at[pl.ds(0, 32 * 16)])
    pltpu.sync_copy(pcnt_hbm.at[pl.ds(c1, 32 * 16)],
                    cs.at[pl.ds(32 * 16, 32 * 16)])
    pltpu.sync_copy(wbm_hbm, wbm)

    half = jnp.full((16,), 0.5, jnp.float32)
    three_half = jnp.full((16,), 1.5, jnp.float32)
    magic = jnp.full((16,), 0x5f3759df, jnp.int32)

    @plsc.parallel_loop(0, 32, step=1, unroll=2)
    def _seg(k):
        cnt = cs[pl.ds(k * 16, 16)] + cs[pl.ds(32 * 16 + k * 16, 16)]
        n = jnp.maximum(cnt, 1.0)
        inv_n = 1.0 / n
        for j in range(_NJ):
            sl = pl.ds(k * D + 16 * j, 16)
            sl2 = pl.ds(32 * D + k * D + 16 * j, 16)
            w = wbm[pl.ds(16 * j, 16)]
            bia = wbm[pl.ds(D + 16 * j, 16)]
            msf = wbm[pl.ds(2 * D + 16 * j, 16)]
            su = ps[sl] + ps[sl2]
            sq = qs[sl] + qs[sl2]
            m = su * inv_n
            msc = m * msf
            varsum = jnp.maximum(sq - msc * (2.0 * su - n * msc), 0.0)
            x = varsum * inv_n + EPS
            xi = lax.bitcast_convert_type(x, jnp.int32)
            y = lax.bitcast_convert_type(magic - (xi >> 1), jnp.float32)
            hx = half * x
            for _i in range(3):
                y = y * (three_half - hx * y * y)
            a = w * y
            sa[pl.ds(k * D + 16 * j, 16)] = a
            sc_[pl.ds(k * D + 16 * j, 16)] = bia - a * msc

    t0 = pl.multiple_of((32 * D) * s, 8)
    pltpu.sync_copy(sa, sh_a.at[pl.ds(t0, 32 * D)])
    pltpu.sync_copy(sc_, sh_c.at[pl.ds(t0, 32 * D)])
    plsc.subcore_barrier()

    sb0, sb1, base_al = _worker_range(gid)
    nblk = sb1 - sb0
    pltpu.sync_copy(ids_hbm.at[pl.ds(base_al, IDS_LEN)], ids_v)

    def _in_copy(b, slot):
        boff = pl.multiple_of((sb0 + b) * BLKW, 8)
        doff = pl.multiple_of(slot * BLKW, 8)
        return pltpu.make_async_copy(feats_hbm.at[pl.ds(boff, BLKW)],
                                     buf.at[pl.ds(doff, BLKW)], sem_in)

    def _out_copy(b, slot):
        boff = pl.multiple_of((sb0 + b) * BLKW, 8)
        soff = pl.multiple_of(slot * BLKW, 8)
        return pltpu.make_async_copy(obuf.at[pl.ds(soff, BLKW)],
                                     out_hbm.at[pl.ds(boff, BLKW)], sem_out)

    _in_copy(0, 0).start()

    def _pull(sid):
        soff = pl.multiple_of(sid * D, 8)
        pltpu.sync_copy(sh_a.at[pl.ds(soff, D)], arow)
        pltpu.sync_copy(sh_c.at[pl.ds(soff, D)], crow)

    def _block(b, cur):
        _in_copy(b, b % 2).wait()

        @pl.when(b + 1 < nblk)
        def _():
            _in_copy(b + 1, (b + 1) % 2).start()

        @pl.when(b >= 2)
        def _():
            _out_copy(b - 2, b % 2).wait()

        bb = pl.multiple_of((b % 2) * BLKW, 8)

        def _group(g, cur):
            row0 = b * BLK + g * 16
            idv = ids_v[pl.ds(row0, 16)]
            first = idv[0]
            last = idv[15]
            uniform = first == last

            @pl.when(uniform & (first != cur))
            def _():
                _pull(first)

            def _fast(cur):
                avs = [arow[pl.ds(16 * j, 16)] for j in range(_NJ)]
                cvs = [crow[pl.ds(16 * j, 16)] for j in range(_NJ)]

                @plsc.parallel_loop(0, 16, step=1, unroll=4)
                def _rows(r):
                    o = bb + (g * 16 + r) * D
                    for j in range(_NJ):
                        x = buf[pl.ds(o + 16 * j, 16)]
                        obuf[pl.ds(o + 16 * j, 16)] = avs[j] * x + cvs[j]
                return last

            def _slow(cur):
                def _srow(r, cur):
                    sid = ids_v[pl.ds(row0 + r, 16)][0]

                    @pl.when(sid != cur)
                    def _():
                        _pull(sid)

                    o = bb + (g * 16 + r) * D
                    for j in range(_NJ):
                        x = buf[pl.ds(o + 16 * j, 16)]
                        obuf[pl.ds(o + 16 * j, 16)] = (
                            arow[pl.ds(16 * j, 16)] * x
                            + crow[pl.ds(16 * j, 16)])
                    return jnp.where(sid != cur, sid, cur)
                return lax.fori_loop(0, 16, _srow, cur)

            return lax.cond(uniform, _fast, _slow, cur)

        cur = lax.fori_loop(0, NG, _group, cur)
        _out_copy(b, b % 2).start()
        return cur

    lax.fori_loop(0, nblk, _block, jnp.int32(-1))

    @pl.when(nblk >= 2)
    def _():
        _out_copy(nblk - 2, nblk % 2).wait()
    _out_copy(nblk - 1, (nblk - 1) % 2).wait()


def kernel(feats, segment_ids, weight, bias, mean_scale):
    ids = segment_ids.astype(jnp.int32)
    ids_pad = jnp.pad(ids, (0, 128))
    feats_flat = feats.reshape(N * D)
    psum, psq, pcnt = _sc_accumulate(feats_flat, ids_pad)
    wbm = jnp.concatenate([weight, bias, mean_scale])
    out = _sc_normalize(feats_flat, ids_pad, psum, psq, pcnt, wbm)
    return out.reshape(N, D)
